# baseline (speedup 1.0000x reference)
import jax
import jax.numpy as jnp
from jax.experimental import pallas as pl
from jax.experimental.pallas import tpu as pltpu

_NCHUNK = 1


def _copy_pipelined(src_ref, dst_ref, buf, rsem, wsem):
    m = src_ref.shape[0]
    blk = m // _NCHUNK

    def rcopy(j):
        return pltpu.make_async_copy(
            src_ref.at[pl.ds(j * blk, blk), :], buf.at[j], rsem.at[j])

    def wcopy(j):
        return pltpu.make_async_copy(
            buf.at[j], dst_ref.at[pl.ds(j * blk, blk), :], wsem.at[j])

    for j in range(_NCHUNK):
        rcopy(j).start()
    for j in range(_NCHUNK):
        rcopy(j).wait()
        wcopy(j).start()
    for j in range(_NCHUNK):
        wcopy(j).wait()


def kernel(x, pos_emb):
    del x
    m, d = pos_emb.shape
    blk = m // _NCHUNK
    return pl.pallas_call(
        _copy_pipelined,
        in_specs=[pl.BlockSpec(memory_space=pltpu.MemorySpace.HBM)],
        out_specs=pl.BlockSpec(memory_space=pltpu.MemorySpace.HBM),
        scratch_shapes=[
            pltpu.VMEM((_NCHUNK, blk, d), jnp.float32),
            pltpu.SemaphoreType.DMA((_NCHUNK,)),
            pltpu.SemaphoreType.DMA((_NCHUNK,)),
        ],
        out_shape=jax.ShapeDtypeStruct((m, d), pos_emb.dtype),
    )(pos_emb)


# speedup vs baseline: 1.0524x; 1.0524x over previous
import jax
import jax.numpy as jnp
from jax.experimental import pallas as pl

_BLK = 4096


def _copy_block(src_ref, dst_ref):
    dst_ref[...] = src_ref[...]


def kernel(x, pos_emb):
    del x
    m, d = pos_emb.shape
    return pl.pallas_call(
        _copy_block,
        grid=(m // _BLK,),
        in_specs=[pl.BlockSpec((_BLK, d), lambda i: (i, 0))],
        out_specs=pl.BlockSpec((_BLK, d), lambda i: (i, 0)),
        out_shape=jax.ShapeDtypeStruct((m, d), pos_emb.dtype),
    )(pos_emb)
